# baseline (speedup 1.0000x reference)
import jax
import jax.numpy as jnp
from jax.experimental import pallas as pl
from jax.experimental.pallas import tpu as pltpu

_B, _N, _M, _D = 8, 4096, 4096, 3
_TILE_N = 2048
_TILE_M = 512
_NI = _N // _TILE_N
_MI = _M // _TILE_M
_SENTINEL = 1e30


def _nn_loss_kernel(pp_ref, tp_ref, pred_ref, tgt_ref,
                    total_ref, coord_ref, pts_ref,
                    rhs_ref, csum_ref):
    b = pl.program_id(0)
    ni = pl.program_id(1)

    pcount = pp_ref[b]
    tcount = tp_ref[b]
    rows = jax.lax.broadcasted_iota(jnp.int32, (4, 1), 0)

    @pl.when(jnp.logical_and(b == 0, ni == 0))
    def _():
        csum_ref[0, 0] = jnp.float32(0.0)

    @pl.when(ni == 0)
    def _():
        t = tgt_ref[0]
        tnorm = jnp.sum(t * t, axis=0, keepdims=True)
        j = jax.lax.broadcasted_iota(jnp.int32, (1, _M), 1)
        tnorm = jnp.where(j < tcount, tnorm, jnp.float32(_SENTINEL))
        rhs = jnp.where(rows == 3, tnorm, t)
        rhs_hi = rhs.astype(jnp.bfloat16)
        rhs_lo = (rhs - rhs_hi.astype(jnp.float32)).astype(jnp.bfloat16)
        rhs_ref[...] = jnp.concatenate([rhs_hi, rhs_lo, rhs_hi], axis=0)

    @pl.when(ni * _TILE_N < pcount)
    def _():
        p = pred_ref[0]
        lhs = jnp.where(rows == 3, jnp.float32(1.0), -2.0 * p)
        lhs_hi = lhs.astype(jnp.bfloat16)
        lhs_lo = (lhs - lhs_hi.astype(jnp.float32)).astype(jnp.bfloat16)
        lhs12 = jnp.concatenate([lhs_hi, lhs_hi, lhs_lo], axis=0)

        def one_tile(mi, m):
            rhs12 = rhs_ref[:, pl.ds(mi * _TILE_M, _TILE_M)]
            cand = jax.lax.dot_general(
                rhs12, lhs12, (((0,), (0,)), ((), ())),
                preferred_element_type=jnp.float32)
            return jnp.minimum(m, jnp.min(cand, axis=0))

        def pair_body(pi, m):
            return one_tile(2 * pi + 1, one_tile(2 * pi, m))

        n_tiles = (tcount + (_TILE_M - 1)) // _TILE_M
        init = jnp.full((_TILE_N,), jnp.inf, jnp.float32)
        m = jax.lax.fori_loop(0, n_tiles, one_tile, init)

        rowmin = m
        pnorm = jnp.sum(p * p, axis=0)
        i = ni * _TILE_N + jax.lax.iota(jnp.int32, _TILE_N)
        vals = jnp.where(i < pcount, rowmin + pnorm, jnp.float32(0.0))
        s = jnp.sum(vals) / (pcount.astype(jnp.float32) * jnp.float32(_D))
        csum_ref[0, 0] = csum_ref[0, 0] + s

    @pl.when(jnp.logical_and(b == _B - 1, ni == _NI - 1))
    def _():
        coord = csum_ref[0, 0] / jnp.float32(_B)
        pts = jnp.float32(0.0)
        for bb in range(_B):
            dv = (pp_ref[bb] - tp_ref[bb]).astype(jnp.float32) / _N
            pts = pts + dv * dv
        pts = pts / jnp.float32(_B)
        coord_ref[0, 0] = coord
        pts_ref[0, 0] = pts
        total_ref[0, 0] = coord + jnp.float32(0.1) * pts


def kernel(predicted_coords, predicted_points, target_coords, target_points):
    pp = predicted_points.astype(jnp.int32)
    tp = target_points.astype(jnp.int32)
    pred_t = jnp.pad(jnp.transpose(predicted_coords, (0, 2, 1)),
                     ((0, 0), (0, 4 - _D), (0, 0)))
    tgt_t = jnp.pad(jnp.transpose(target_coords, (0, 2, 1)),
                    ((0, 0), (0, 4 - _D), (0, 0)))

    out_shape = [jax.ShapeDtypeStruct((1, 1), jnp.float32)] * 3
    total, coord, pts = pl.pallas_call(
        _nn_loss_kernel,
        grid=(_B, _NI),
        in_specs=[
            pl.BlockSpec(memory_space=pltpu.SMEM),
            pl.BlockSpec(memory_space=pltpu.SMEM),
            pl.BlockSpec((1, 4, _TILE_N), lambda b, ni: (b, 0, ni)),
            pl.BlockSpec((1, 4, _M), lambda b, ni: (b, 0, 0)),
        ],
        out_specs=[
            pl.BlockSpec(memory_space=pltpu.SMEM),
            pl.BlockSpec(memory_space=pltpu.SMEM),
            pl.BlockSpec(memory_space=pltpu.SMEM),
        ],
        out_shape=out_shape,
        scratch_shapes=[
            pltpu.VMEM((12, _M), jnp.bfloat16),
            pltpu.SMEM((1, 1), jnp.float32),
        ],
        compiler_params=pltpu.CompilerParams(
            dimension_semantics=("arbitrary", "arbitrary")),
    )(pp, tp, pred_t, tgt_t)
    return total[0, 0], coord[0, 0], pts[0, 0]


# speedup vs baseline: 1.2016x; 1.2016x over previous
import jax
import jax.numpy as jnp
from jax.experimental import pallas as pl
from jax.experimental.pallas import tpu as pltpu

_B, _N, _M, _D = 8, 4096, 4096, 3
_TILE_N = 2048
_TILE_M = 512
_NI = _N // _TILE_N
_MI = _M // _TILE_M
_SENTINEL = 1e30


def _nn_loss_kernel(pp_ref, tp_ref, pred_ref, tgt_ref,
                    total_ref, coord_ref, pts_ref,
                    rhs_ref, csum_ref):
    b = pl.program_id(0)
    ni = pl.program_id(1)

    pcount = pp_ref[b]
    tcount = tp_ref[b]
    rows = jax.lax.broadcasted_iota(jnp.int32, (4, 1), 0)

    @pl.when(jnp.logical_and(b == 0, ni == 0))
    def _():
        csum_ref[0, 0] = jnp.float32(0.0)

    @pl.when(ni == 0)
    def _():
        t = tgt_ref[0]
        tnorm = jnp.sum(t * t, axis=0, keepdims=True)
        j = jax.lax.broadcasted_iota(jnp.int32, (1, _M), 1)
        tnorm = jnp.where(j < tcount, tnorm, jnp.float32(_SENTINEL))
        rhs = jnp.where(rows == 3, tnorm, t)
        rhs_hi = rhs.astype(jnp.bfloat16)
        rhs_lo = (rhs - rhs_hi.astype(jnp.float32)).astype(jnp.bfloat16)
        rhs_ref[...] = jnp.concatenate([rhs_hi, rhs_lo, rhs_hi], axis=0)

    @pl.when(ni * _TILE_N < pcount)
    def _():
        p = pred_ref[0]
        lhs = jnp.where(rows == 3, jnp.float32(1.0), -2.0 * p)
        lhs_hi = lhs.astype(jnp.bfloat16)
        lhs_lo = (lhs - lhs_hi.astype(jnp.float32)).astype(jnp.bfloat16)
        lhs12 = jnp.concatenate([lhs_hi, lhs_hi, lhs_lo], axis=0)

        def one_tile(mi, m):
            rhs12 = rhs_ref[:, mi * _TILE_M:(mi + 1) * _TILE_M]
            cand = jax.lax.dot_general(
                rhs12, lhs12, (((0,), (0,)), ((), ())),
                preferred_element_type=jnp.float32)
            return jnp.minimum(m, jnp.min(cand, axis=0))

        def make_branch(k):
            def branch(mm):
                for mi in range(k + 1):
                    mm = one_tile(mi, mm)
                return mm
            return branch

        n_tiles = (tcount + (_TILE_M - 1)) // _TILE_M
        init = jnp.full((_TILE_N,), jnp.inf, jnp.float32)
        m = jax.lax.switch(n_tiles - 1,
                           [make_branch(k) for k in range(_MI)], init)

        rowmin = m
        pnorm = jnp.sum(p * p, axis=0)
        i = ni * _TILE_N + jax.lax.iota(jnp.int32, _TILE_N)
        vals = jnp.where(i < pcount, rowmin + pnorm, jnp.float32(0.0))
        s = jnp.sum(vals) / (pcount.astype(jnp.float32) * jnp.float32(_D))
        csum_ref[0, 0] = csum_ref[0, 0] + s

    @pl.when(jnp.logical_and(b == _B - 1, ni == _NI - 1))
    def _():
        coord = csum_ref[0, 0] / jnp.float32(_B)
        pts = jnp.float32(0.0)
        for bb in range(_B):
            dv = (pp_ref[bb] - tp_ref[bb]).astype(jnp.float32) / _N
            pts = pts + dv * dv
        pts = pts / jnp.float32(_B)
        coord_ref[0, 0] = coord
        pts_ref[0, 0] = pts
        total_ref[0, 0] = coord + jnp.float32(0.1) * pts


def kernel(predicted_coords, predicted_points, target_coords, target_points):
    pp = predicted_points.astype(jnp.int32)
    tp = target_points.astype(jnp.int32)
    pred_t = jnp.pad(jnp.transpose(predicted_coords, (0, 2, 1)),
                     ((0, 0), (0, 4 - _D), (0, 0)))
    tgt_t = jnp.pad(jnp.transpose(target_coords, (0, 2, 1)),
                    ((0, 0), (0, 4 - _D), (0, 0)))

    out_shape = [jax.ShapeDtypeStruct((1, 1), jnp.float32)] * 3
    total, coord, pts = pl.pallas_call(
        _nn_loss_kernel,
        grid=(_B, _NI),
        in_specs=[
            pl.BlockSpec(memory_space=pltpu.SMEM),
            pl.BlockSpec(memory_space=pltpu.SMEM),
            pl.BlockSpec((1, 4, _TILE_N), lambda b, ni: (b, 0, ni)),
            pl.BlockSpec((1, 4, _M), lambda b, ni: (b, 0, 0)),
        ],
        out_specs=[
            pl.BlockSpec(memory_space=pltpu.SMEM),
            pl.BlockSpec(memory_space=pltpu.SMEM),
            pl.BlockSpec(memory_space=pltpu.SMEM),
        ],
        out_shape=out_shape,
        scratch_shapes=[
            pltpu.VMEM((12, _M), jnp.bfloat16),
            pltpu.SMEM((1, 1), jnp.float32),
        ],
        compiler_params=pltpu.CompilerParams(
            dimension_semantics=("arbitrary", "arbitrary")),
    )(pp, tp, pred_t, tgt_t)
    return total[0, 0], coord[0, 0], pts[0, 0]
